# baseline (speedup 1.0000x reference)
import functools

import jax
import jax.numpy as jnp
from jax import lax
from jax.experimental import pallas as pl
from jax.experimental.pallas import tpu as pltpu
from jax.experimental.pallas import tpu_sc as plsc

N = 10000
E = 160000
D = 256
R = 256
P = 64

_NC = 2
_NS = 16
_NW = _NC * _NS
_EPW = E // _NW
_C = 256
_NFULL = _EPW // _C
_CT = _EPW - _NFULL * _C
_EH = E // 2

_HI = -65536
_LO = 65535
_RND = 0x8000


def _node_body(nf_ref, wn_ref, bn_ref, wsrc_ref, wdst_ref, tbl_ref):
    n = jnp.maximum(
        jnp.dot(nf_ref[...], wn_ref[...], preferred_element_type=jnp.float32)
        + bn_ref[...], 0.0)
    q = jnp.dot(n, wsrc_ref[...], preferred_element_type=jnp.float32)
    k = jnp.dot(n, wdst_ref[...], preferred_element_type=jnp.float32)
    q32 = lax.bitcast_convert_type(q, jnp.int32)
    k32 = lax.bitcast_convert_type(k, jnp.int32)
    tbl_ref[...] = ((q32 + _RND) & _HI) | lax.shift_right_logical(
        k32 + _RND, 16)


def _node_tc(nf, Wn, bn, Wsrc, Wdst):
    blk = 2000
    return pl.pallas_call(
        _node_body,
        grid=(N // blk,),
        in_specs=[
            pl.BlockSpec((blk, D), lambda i: (i, 0)),
            pl.BlockSpec((D, R), lambda i: (0, 0)),
            pl.BlockSpec((1, R), lambda i: (0, 0)),
            pl.BlockSpec((R, P), lambda i: (0, 0)),
            pl.BlockSpec((R, P), lambda i: (0, 0)),
        ],
        out_specs=pl.BlockSpec((blk, P), lambda i: (i, 0)),
        out_shape=jax.ShapeDtypeStruct((N, P), jnp.int32),
    )(nf, Wn, bn.reshape(1, R), Wsrc, Wdst)


def _sc_gather_body(tbl_hbm, src_hbm, dst_hbm, out_hbm,
                    sidx_all, didx_all,
                    a_a, b_a, p_a, a_b, b_b, p_b,
                    sg_a, sg_b, sw_a, sw_b, sem_t):
    wid = lax.axis_index("s") * _NC + lax.axis_index("c")
    base_w = pl.multiple_of(wid * _EPW, 8)
    row_w = pl.multiple_of((wid % (_NW // 2)) * _EPW, 8)
    col_w = pl.multiple_of((wid // (_NW // 2)) * P, P)

    pltpu.sync_copy(src_hbm.at[pl.ds(base_w, _EPW)], sidx_all)
    pltpu.sync_copy(dst_hbm.at[pl.ds(base_w, _EPW)], didx_all)

    def issue_gather(c, av, bv, sem):
        off = pl.multiple_of(c * _C, 8)
        ca = pltpu.async_copy(tbl_hbm.at[sidx_all.at[pl.ds(off, _C)]], av, sem)
        cb = pltpu.async_copy(tbl_hbm.at[didx_all.at[pl.ds(off, _C)]], bv, sem)
        return ca, cb

    def pack(c_rows, av, bv, pv):
        def row_body(i, _):
            for s in range(P // 16):
                sl = (i, pl.ds(s * 16, 16))
                pv[sl] = (av[sl] & _HI) | (bv[sl] & _LO)
            return 0
        lax.fori_loop(0, c_rows, row_body, 0, unroll=8)

    def issue_writeback(c, pv, sem):
        row = pl.multiple_of(row_w + c * _C, 8)
        return pltpu.async_copy(
            pv, out_hbm.at[pl.ds(row, _C), pl.ds(col_w, P)], sem)

    def pair_body(i, _):
        c0 = 2 * i
        ga = issue_gather(c0, a_a, b_a, sg_a)
        gb = issue_gather(c0 + 1, a_b, b_b, sg_b)
        ga[0].wait()
        ga[1].wait()
        pack(_C, a_a, b_a, p_a)
        wa = issue_writeback(c0, p_a, sw_a)
        gb[0].wait()
        gb[1].wait()
        pack(_C, a_b, b_b, p_b)
        wb = issue_writeback(c0 + 1, p_b, sw_b)
        wa.wait()
        wb.wait()
        return 0

    lax.fori_loop(0, _NFULL // 2, pair_body, 0)

    ga = issue_gather(_NFULL - 1, a_a, b_a, sg_a)
    toff = pl.multiple_of(_NFULL * _C, 8)
    ca = pltpu.async_copy(tbl_hbm.at[sidx_all.at[pl.ds(toff, _CT)]],
                          a_b.at[pl.ds(0, _CT), :], sem_t)
    cb = pltpu.async_copy(tbl_hbm.at[didx_all.at[pl.ds(toff, _CT)]],
                          b_b.at[pl.ds(0, _CT), :], sem_t)
    ga[0].wait()
    ga[1].wait()
    pack(_C, a_a, b_a, p_a)
    wa = issue_writeback(_NFULL - 1, p_a, sw_a)
    ca.wait()
    cb.wait()
    pack(_CT, a_b, b_b, p_b)
    trow = pl.multiple_of(row_w + _NFULL * _C, 8)
    pltpu.sync_copy(p_b.at[pl.ds(0, _CT), :],
                    out_hbm.at[pl.ds(trow, _CT), pl.ds(col_w, P)])
    wa.wait()


def _sc_gather(tbl, src, dst):
    mesh = plsc.VectorSubcoreMesh(core_axis_name="c", subcore_axis_name="s")
    kern = functools.partial(
        pl.kernel,
        out_type=jax.ShapeDtypeStruct((_EH, 2 * P), jnp.int32),
        mesh=mesh,
        scratch_types=[
            pltpu.VMEM((_EPW,), jnp.int32),
            pltpu.VMEM((_EPW,), jnp.int32),
            pltpu.VMEM((_C, P), jnp.int32),
            pltpu.VMEM((_C, P), jnp.int32),
            pltpu.VMEM((_C, P), jnp.int32),
            pltpu.VMEM((_C, P), jnp.int32),
            pltpu.VMEM((_C, P), jnp.int32),
            pltpu.VMEM((_C, P), jnp.int32),
            pltpu.SemaphoreType.DMA,
            pltpu.SemaphoreType.DMA,
            pltpu.SemaphoreType.DMA,
            pltpu.SemaphoreType.DMA,
            pltpu.SemaphoreType.DMA,
        ],
        compiler_params=pltpu.CompilerParams(use_tc_tiling_on_sc=False),
    )(_sc_gather_body)
    return kern(tbl, src, dst)


_EBLK = 3200
_NSTEP = _EH // _EBLK


def _edge_body(efa_ref, efb_ref, we_ref, be_ref, wedge2_ref, qskd_ref,
               sa_ref, sb_ref):
    we = we_ref[...]
    be = be_ref[...]
    wedge2 = wedge2_ref[...]

    ea = jnp.maximum(
        jnp.dot(efa_ref[...], we, preferred_element_type=jnp.float32)
        + be, 0.0)
    epw_a = jnp.dot(ea, wedge2, preferred_element_type=jnp.float32)
    eb = jnp.maximum(
        jnp.dot(efb_ref[...], we, preferred_element_type=jnp.float32)
        + be, 0.0)
    epw_b = jnp.dot(eb, wedge2, preferred_element_type=jnp.float32)

    lane = lax.broadcasted_iota(jnp.int32, (_EBLK, 2 * P), 1)
    in_a = lane < P
    ep128 = jnp.where(in_a, epw_a, epw_b)

    x = qskd_ref[...]
    qs = lax.bitcast_convert_type(x & _HI, jnp.float32)
    kd = lax.bitcast_convert_type(lax.shift_left(x, 16), jnp.float32)
    score128 = qs * kd + ep128 * (qs + kd)

    zero = jnp.zeros_like(score128)
    sa = jnp.sum(jnp.where(in_a, score128, zero), axis=-1)
    sb = jnp.sum(jnp.where(in_a, zero, score128), axis=-1)
    sa_ref[...] = sa.reshape(1, _EBLK // 128, 128)
    sb_ref[...] = sb.reshape(1, _EBLK // 128, 128)


def _edge_tc(ef, We, be, Wedge2, qskd):
    nrow = _EBLK // 128
    sa, sb = pl.pallas_call(
        _edge_body,
        grid=(_NSTEP,),
        in_specs=[
            pl.BlockSpec((_EBLK, D), lambda i: (i, 0)),
            pl.BlockSpec((_EBLK, D), lambda i: (i + _NSTEP, 0)),
            pl.BlockSpec((D, R), lambda i: (0, 0)),
            pl.BlockSpec((1, R), lambda i: (0, 0)),
            pl.BlockSpec((R, 2 * P), lambda i: (0, 0)),
            pl.BlockSpec((_EBLK, 2 * P), lambda i: (i, 0)),
        ],
        out_specs=[
            pl.BlockSpec((1, nrow, 128), lambda i: (i, 0, 0)),
            pl.BlockSpec((1, nrow, 128), lambda i: (i, 0, 0)),
        ],
        out_shape=[jax.ShapeDtypeStruct((_NSTEP, nrow, 128), jnp.float32)] * 2,
    )(ef, ef, We, be.reshape(1, R), Wedge2, qskd)
    return jnp.concatenate([sa.reshape(_EH), sb.reshape(_EH)])


def kernel(node_features, edge_features, edge_index, Wn, bn, We, be,
           Wsrc, Wdst, Wedge):
    tbl = _node_tc(node_features, Wn, bn, Wsrc, Wdst)
    src = edge_index[0].astype(jnp.int32)
    dst = edge_index[1].astype(jnp.int32)
    qskd = _sc_gather(tbl, src, dst)
    wedge2 = jnp.concatenate([Wedge, Wedge], axis=1)
    return _edge_tc(edge_features, We, be, wedge2, qskd)


# speedup vs baseline: 1.4279x; 1.4279x over previous
import functools

import jax
import jax.numpy as jnp
from jax import lax
from jax.experimental import pallas as pl
from jax.experimental.pallas import tpu as pltpu
from jax.experimental.pallas import tpu_sc as plsc

N = 10000
E = 160000
D = 256
R = 256
P = 64

_NC = 2
_NS = 16
_NW = _NC * _NS
_EPW = E // _NW
_C = 256
_NFULL = _EPW // _C
_CT = _EPW - _NFULL * _C
_EH = E // 2

_HI = -65536
_LO = 65535
_RND = 0x8000


def _node_body(nf_ref, wn_ref, bn_ref, wsrc_ref, wdst_ref, tbl_ref):
    n = jnp.maximum(
        jnp.dot(nf_ref[...], wn_ref[...], preferred_element_type=jnp.float32)
        + bn_ref[...], 0.0)
    q = jnp.dot(n, wsrc_ref[...], preferred_element_type=jnp.float32)
    k = jnp.dot(n, wdst_ref[...], preferred_element_type=jnp.float32)
    q32 = lax.bitcast_convert_type(q, jnp.int32)
    k32 = lax.bitcast_convert_type(k, jnp.int32)
    tbl_ref[...] = ((q32 + _RND) & _HI) | lax.shift_right_logical(
        k32 + _RND, 16)


def _node_tc(nf, Wn, bn, Wsrc, Wdst):
    blk = 2000
    return pl.pallas_call(
        _node_body,
        grid=(N // blk,),
        in_specs=[
            pl.BlockSpec((blk, D), lambda i: (i, 0)),
            pl.BlockSpec((D, R), lambda i: (0, 0)),
            pl.BlockSpec((1, R), lambda i: (0, 0)),
            pl.BlockSpec((R, P), lambda i: (0, 0)),
            pl.BlockSpec((R, P), lambda i: (0, 0)),
        ],
        out_specs=pl.BlockSpec((blk, P), lambda i: (i, 0)),
        out_shape=jax.ShapeDtypeStruct((N, P), jnp.int32),
    )(nf, Wn, bn.reshape(1, R), Wsrc, Wdst)


def _sc_gather_body(tbl_hbm, src_hbm, dst_hbm, out_hbm,
                    sidx_all, didx_all,
                    a_a, b_a, p_a, a_b, b_b, p_b,
                    sg_a, sg_b, sw_a, sw_b, sem_t):
    wid = lax.axis_index("s") * _NC + lax.axis_index("c")
    base_w = pl.multiple_of(wid * _EPW, 8)
    row_w = pl.multiple_of((wid % (_NW // 2)) * _EPW, 8)
    col_w = pl.multiple_of((wid // (_NW // 2)) * P, P)

    pltpu.sync_copy(src_hbm.at[pl.ds(base_w, _EPW)], sidx_all)
    pltpu.sync_copy(dst_hbm.at[pl.ds(base_w, _EPW)], didx_all)

    def issue_gather(c, av, bv, sem):
        off = pl.multiple_of(c * _C, 8)
        ca = pltpu.async_copy(tbl_hbm.at[sidx_all.at[pl.ds(off, _C)]], av, sem)
        cb = pltpu.async_copy(tbl_hbm.at[didx_all.at[pl.ds(off, _C)]], bv, sem)
        return ca, cb

    def pack(c_rows, av, bv, pv):
        @plsc.parallel_loop(0, c_rows, 1, unroll=8)
        def _(i):
            for s in range(P // 16):
                sl = (i, pl.ds(s * 16, 16))
                pv[sl] = (av[sl] & _HI) | (bv[sl] & _LO)

    def issue_writeback(c, pv, sem):
        row = pl.multiple_of(row_w + c * _C, 8)
        return pltpu.async_copy(
            pv, out_hbm.at[pl.ds(row, _C), pl.ds(col_w, P)], sem)

    def pair_body(i, _):
        c0 = 2 * i
        ga = issue_gather(c0, a_a, b_a, sg_a)
        gb = issue_gather(c0 + 1, a_b, b_b, sg_b)
        ga[0].wait()
        ga[1].wait()
        pack(_C, a_a, b_a, p_a)
        wa = issue_writeback(c0, p_a, sw_a)
        gb[0].wait()
        gb[1].wait()
        pack(_C, a_b, b_b, p_b)
        wb = issue_writeback(c0 + 1, p_b, sw_b)
        wa.wait()
        wb.wait()
        return 0

    lax.fori_loop(0, _NFULL // 2, pair_body, 0)

    ga = issue_gather(_NFULL - 1, a_a, b_a, sg_a)
    toff = pl.multiple_of(_NFULL * _C, 8)
    ca = pltpu.async_copy(tbl_hbm.at[sidx_all.at[pl.ds(toff, _CT)]],
                          a_b.at[pl.ds(0, _CT), :], sem_t)
    cb = pltpu.async_copy(tbl_hbm.at[didx_all.at[pl.ds(toff, _CT)]],
                          b_b.at[pl.ds(0, _CT), :], sem_t)
    ga[0].wait()
    ga[1].wait()
    pack(_C, a_a, b_a, p_a)
    wa = issue_writeback(_NFULL - 1, p_a, sw_a)
    ca.wait()
    cb.wait()
    pack(_CT, a_b, b_b, p_b)
    trow = pl.multiple_of(row_w + _NFULL * _C, 8)
    pltpu.sync_copy(p_b.at[pl.ds(0, _CT), :],
                    out_hbm.at[pl.ds(trow, _CT), pl.ds(col_w, P)])
    wa.wait()


def _sc_gather(tbl, src, dst):
    mesh = plsc.VectorSubcoreMesh(core_axis_name="c", subcore_axis_name="s")
    kern = functools.partial(
        pl.kernel,
        out_type=jax.ShapeDtypeStruct((_EH, 2 * P), jnp.int32),
        mesh=mesh,
        scratch_types=[
            pltpu.VMEM((_EPW,), jnp.int32),
            pltpu.VMEM((_EPW,), jnp.int32),
            pltpu.VMEM((_C, P), jnp.int32),
            pltpu.VMEM((_C, P), jnp.int32),
            pltpu.VMEM((_C, P), jnp.int32),
            pltpu.VMEM((_C, P), jnp.int32),
            pltpu.VMEM((_C, P), jnp.int32),
            pltpu.VMEM((_C, P), jnp.int32),
            pltpu.SemaphoreType.DMA,
            pltpu.SemaphoreType.DMA,
            pltpu.SemaphoreType.DMA,
            pltpu.SemaphoreType.DMA,
            pltpu.SemaphoreType.DMA,
        ],
        compiler_params=pltpu.CompilerParams(use_tc_tiling_on_sc=False),
    )(_sc_gather_body)
    return kern(tbl, src, dst)


_EBLK = 3200
_NSTEP = _EH // _EBLK


def _edge_body(efa_ref, efb_ref, we_ref, be_ref, wedge2_ref, qskd_ref,
               sa_ref, sb_ref):
    we = we_ref[...]
    be = be_ref[...]
    wedge2 = wedge2_ref[...]

    ea = jnp.maximum(
        jnp.dot(efa_ref[...], we, preferred_element_type=jnp.float32)
        + be, 0.0)
    epw_a = jnp.dot(ea, wedge2, preferred_element_type=jnp.float32)
    eb = jnp.maximum(
        jnp.dot(efb_ref[...], we, preferred_element_type=jnp.float32)
        + be, 0.0)
    epw_b = jnp.dot(eb, wedge2, preferred_element_type=jnp.float32)

    lane = lax.broadcasted_iota(jnp.int32, (_EBLK, 2 * P), 1)
    in_a = lane < P
    ep128 = jnp.where(in_a, epw_a, epw_b)

    x = qskd_ref[...]
    qs = lax.bitcast_convert_type(x & _HI, jnp.float32)
    kd = lax.bitcast_convert_type(lax.shift_left(x, 16), jnp.float32)
    score128 = qs * kd + ep128 * (qs + kd)

    zero = jnp.zeros_like(score128)
    sa = jnp.sum(jnp.where(in_a, score128, zero), axis=-1)
    sb = jnp.sum(jnp.where(in_a, zero, score128), axis=-1)
    sa_ref[...] = sa.reshape(1, _EBLK // 128, 128)
    sb_ref[...] = sb.reshape(1, _EBLK // 128, 128)


def _edge_tc(ef, We, be, Wedge2, qskd):
    nrow = _EBLK // 128
    sa, sb = pl.pallas_call(
        _edge_body,
        grid=(_NSTEP,),
        in_specs=[
            pl.BlockSpec((_EBLK, D), lambda i: (i, 0)),
            pl.BlockSpec((_EBLK, D), lambda i: (i + _NSTEP, 0)),
            pl.BlockSpec((D, R), lambda i: (0, 0)),
            pl.BlockSpec((1, R), lambda i: (0, 0)),
            pl.BlockSpec((R, 2 * P), lambda i: (0, 0)),
            pl.BlockSpec((_EBLK, 2 * P), lambda i: (i, 0)),
        ],
        out_specs=[
            pl.BlockSpec((1, nrow, 128), lambda i: (i, 0, 0)),
            pl.BlockSpec((1, nrow, 128), lambda i: (i, 0, 0)),
        ],
        out_shape=[jax.ShapeDtypeStruct((_NSTEP, nrow, 128), jnp.float32)] * 2,
    )(ef, ef, We, be.reshape(1, R), Wedge2, qskd)
    return jnp.concatenate([sa.reshape(_EH), sb.reshape(_EH)])


def kernel(node_features, edge_features, edge_index, Wn, bn, We, be,
           Wsrc, Wdst, Wedge):
    tbl = _node_tc(node_features, Wn, bn, Wsrc, Wdst)
    src = edge_index[0].astype(jnp.int32)
    dst = edge_index[1].astype(jnp.int32)
    qskd = _sc_gather(tbl, src, dst)
    wedge2 = jnp.concatenate([Wedge, Wedge], axis=1)
    return _edge_tc(edge_features, We, be, wedge2, qskd)


# speedup vs baseline: 1.4827x; 1.0384x over previous
import functools

import jax
import jax.numpy as jnp
from jax import lax
from jax.experimental import pallas as pl
from jax.experimental.pallas import tpu as pltpu
from jax.experimental.pallas import tpu_sc as plsc

N = 10000
E = 160000
D = 256
R = 256
P = 64

_NC = 2
_NS = 16
_NW = _NC * _NS
_EPW = E // _NW
_C = 256
_NFULL = _EPW // _C
_CT = _EPW - _NFULL * _C
_EH = E // 2

_HI = -65536
_LO = 65535
_RND = 0x8000


_NBLK = 1000
_NHS = (N // 2) // _NBLK


def _node_pack(nf, wn, be_, wsrc, wdst):
    n = jnp.maximum(
        jnp.dot(nf, wn, preferred_element_type=jnp.float32) + be_, 0.0)
    q = jnp.dot(n, wsrc, preferred_element_type=jnp.float32)
    k = jnp.dot(n, wdst, preferred_element_type=jnp.float32)
    q32 = lax.bitcast_convert_type(q, jnp.int32)
    k32 = lax.bitcast_convert_type(k, jnp.int32)
    return ((q32 + _RND) & _HI) | lax.shift_right_logical(k32 + _RND, 16)


def _node_body(nfa_ref, nfb_ref, wn_ref, bn_ref, wsrc_ref, wdst_ref, tbl_ref):
    wn = wn_ref[...]
    bn = bn_ref[...]
    wsrc = wsrc_ref[...]
    wdst = wdst_ref[...]
    pa = _node_pack(nfa_ref[...], wn, bn, wsrc, wdst)
    pb = _node_pack(nfb_ref[...], wn, bn, wsrc, wdst)
    tbl_ref[...] = jnp.concatenate([pa, pb], axis=-1)


def _node_tc(nf, Wn, bn, Wsrc, Wdst):
    return pl.pallas_call(
        _node_body,
        grid=(_NHS,),
        in_specs=[
            pl.BlockSpec((_NBLK, D), lambda i: (i, 0)),
            pl.BlockSpec((_NBLK, D), lambda i: (i + _NHS, 0)),
            pl.BlockSpec((D, R), lambda i: (0, 0)),
            pl.BlockSpec((1, R), lambda i: (0, 0)),
            pl.BlockSpec((R, P), lambda i: (0, 0)),
            pl.BlockSpec((R, P), lambda i: (0, 0)),
        ],
        out_specs=pl.BlockSpec((_NBLK, 2 * P), lambda i: (i, 0)),
        out_shape=jax.ShapeDtypeStruct((N // 2, 2 * P), jnp.int32),
    )(nf, nf, Wn, bn.reshape(1, R), Wsrc, Wdst)


def _sc_gather_body(tbl_hbm, src_hbm, dst_hbm, out_hbm,
                    sp_tbl, sidx_all, didx_all,
                    a_a, b_a, a_b, b_b,
                    sg_a, sg_b, sw_a, sw_b, sem_t):
    sid = lax.axis_index("s")
    wid = sid * _NC + lax.axis_index("c")
    base_w = pl.multiple_of(wid * _EPW, 8)
    row_w = pl.multiple_of((wid % (_NW // 2)) * _EPW, 8)
    col_w = pl.multiple_of((wid // (_NW // 2)) * P, P)

    trows = N // _NS
    trow0 = sid * trows
    pltpu.sync_copy(tbl_hbm.at[pl.ds(trow0, trows)],
                    sp_tbl.at[pl.ds(trow0, trows)])
    pltpu.sync_copy(src_hbm.at[pl.ds(base_w, _EPW)], sidx_all)
    pltpu.sync_copy(dst_hbm.at[pl.ds(base_w, _EPW)], didx_all)
    plsc.subcore_barrier()

    def issue_gather(c, av, bv, sem):
        off = pl.multiple_of(c * _C, 8)
        ca = pltpu.async_copy(sp_tbl.at[sidx_all.at[pl.ds(off, _C)]], av, sem)
        cb = pltpu.async_copy(sp_tbl.at[didx_all.at[pl.ds(off, _C)]], bv, sem)
        return ca, cb

    def pack(c_rows, av, bv):
        @plsc.parallel_loop(0, c_rows, 1, unroll=8)
        def _(i):
            for s in range(P // 16):
                sl = (i, pl.ds(s * 16, 16))
                av[sl] = (av[sl] & _HI) | (bv[sl] & _LO)

    def issue_writeback(c, pv, sem):
        row = pl.multiple_of(row_w + c * _C, 8)
        return pltpu.async_copy(
            pv, out_hbm.at[pl.ds(row, _C), pl.ds(col_w, P)], sem)

    def pair_body(i, _):
        c0 = 2 * i
        ga = issue_gather(c0, a_a, b_a, sg_a)
        gb = issue_gather(c0 + 1, a_b, b_b, sg_b)
        ga[0].wait()
        ga[1].wait()
        pack(_C, a_a, b_a)
        wa = issue_writeback(c0, a_a, sw_a)
        gb[0].wait()
        gb[1].wait()
        pack(_C, a_b, b_b)
        wb = issue_writeback(c0 + 1, a_b, sw_b)
        wa.wait()
        wb.wait()
        return 0

    lax.fori_loop(0, _NFULL // 2, pair_body, 0)

    ga = issue_gather(_NFULL - 1, a_a, b_a, sg_a)
    toff = pl.multiple_of(_NFULL * _C, 8)
    ca = pltpu.async_copy(sp_tbl.at[sidx_all.at[pl.ds(toff, _CT)]],
                          a_b.at[pl.ds(0, _CT), :], sem_t)
    cb = pltpu.async_copy(sp_tbl.at[didx_all.at[pl.ds(toff, _CT)]],
                          b_b.at[pl.ds(0, _CT), :], sem_t)
    ga[0].wait()
    ga[1].wait()
    pack(_C, a_a, b_a)
    wa = issue_writeback(_NFULL - 1, a_a, sw_a)
    ca.wait()
    cb.wait()
    pack(_CT, a_b, b_b)
    trow = pl.multiple_of(row_w + _NFULL * _C, 8)
    pltpu.sync_copy(a_b.at[pl.ds(0, _CT), :],
                    out_hbm.at[pl.ds(trow, _CT), pl.ds(col_w, P)])
    wa.wait()


def _sc_gather(tbl, src, dst):
    mesh = plsc.VectorSubcoreMesh(core_axis_name="c", subcore_axis_name="s")
    kern = functools.partial(
        pl.kernel,
        out_type=jax.ShapeDtypeStruct((_EH, 2 * P), jnp.int32),
        mesh=mesh,
        scratch_types=[
            pltpu.VMEM_SHARED((N, P), jnp.int32),
            pltpu.VMEM((_EPW,), jnp.int32),
            pltpu.VMEM((_EPW,), jnp.int32),
            pltpu.VMEM((_C, P), jnp.int32),
            pltpu.VMEM((_C, P), jnp.int32),
            pltpu.VMEM((_C, P), jnp.int32),
            pltpu.VMEM((_C, P), jnp.int32),
            pltpu.SemaphoreType.DMA,
            pltpu.SemaphoreType.DMA,
            pltpu.SemaphoreType.DMA,
            pltpu.SemaphoreType.DMA,
            pltpu.SemaphoreType.DMA,
        ],
        compiler_params=pltpu.CompilerParams(use_tc_tiling_on_sc=False),
    )(_sc_gather_body)
    return kern(tbl, src, dst)


_EBLK = 3200
_NSTEP = _EH // _EBLK


def _edge_body(efa_ref, efb_ref, we_ref, be_ref, wedge2_ref, qskd_ref,
               sa_ref, sb_ref):
    we = we_ref[...]
    be = be_ref[...]
    wedge2 = wedge2_ref[...]

    ea = jnp.maximum(
        jnp.dot(efa_ref[...], we, preferred_element_type=jnp.float32)
        + be, 0.0)
    epw_a = jnp.dot(ea, wedge2, preferred_element_type=jnp.float32)
    eb = jnp.maximum(
        jnp.dot(efb_ref[...], we, preferred_element_type=jnp.float32)
        + be, 0.0)
    epw_b = jnp.dot(eb, wedge2, preferred_element_type=jnp.float32)

    lane = lax.broadcasted_iota(jnp.int32, (_EBLK, 2 * P), 1)
    in_a = lane < P
    ep128 = jnp.where(in_a, epw_a, epw_b)

    x = qskd_ref[...]
    qs = lax.bitcast_convert_type(x & _HI, jnp.float32)
    kd = lax.bitcast_convert_type(lax.shift_left(x, 16), jnp.float32)
    score128 = qs * kd + ep128 * (qs + kd)

    zero = jnp.zeros_like(score128)
    sa = jnp.sum(jnp.where(in_a, score128, zero), axis=-1)
    sb = jnp.sum(jnp.where(in_a, zero, score128), axis=-1)
    sa_ref[...] = sa.reshape(1, _EBLK // 128, 128)
    sb_ref[...] = sb.reshape(1, _EBLK // 128, 128)


def _edge_tc(ef, We, be, Wedge2, qskd):
    nrow = _EBLK // 128
    sa, sb = pl.pallas_call(
        _edge_body,
        grid=(_NSTEP,),
        in_specs=[
            pl.BlockSpec((_EBLK, D), lambda i: (i, 0)),
            pl.BlockSpec((_EBLK, D), lambda i: (i + _NSTEP, 0)),
            pl.BlockSpec((D, R), lambda i: (0, 0)),
            pl.BlockSpec((1, R), lambda i: (0, 0)),
            pl.BlockSpec((R, 2 * P), lambda i: (0, 0)),
            pl.BlockSpec((_EBLK, 2 * P), lambda i: (i, 0)),
        ],
        out_specs=[
            pl.BlockSpec((1, nrow, 128), lambda i: (i, 0, 0)),
            pl.BlockSpec((1, nrow, 128), lambda i: (i, 0, 0)),
        ],
        out_shape=[jax.ShapeDtypeStruct((_NSTEP, nrow, 128), jnp.float32)] * 2,
    )(ef, ef, We, be.reshape(1, R), Wedge2, qskd)
    return jnp.concatenate([sa.reshape(_EH), sb.reshape(_EH)])


def _tbl_row(u):
    return jnp.where(u < N // 2, 2 * u, 2 * u - (N - 1))


def kernel(node_features, edge_features, edge_index, Wn, bn, We, be,
           Wsrc, Wdst, Wedge):
    tbl = _node_tc(node_features, Wn, bn, Wsrc, Wdst).reshape(N, P)
    ei = edge_index.astype(jnp.int32)
    src = _tbl_row(ei[0])
    dst = _tbl_row(ei[1])
    qskd = _sc_gather(tbl, src, dst)
    wedge2 = jnp.concatenate([Wedge, Wedge], axis=1)
    return _edge_tc(edge_features, We, be, wedge2, qskd)
